# Initial kernel scaffold; baseline (speedup 1.0000x reference)
import functools

import jax
import jax.numpy as jnp
from jax import lax
from jax.experimental import pallas as pl
from jax.experimental.pallas import tpu as pltpu
from jax.experimental.pallas import tpu_sc as plsc

_N = 10000
_E = 320000
_D_IN = 128
_D_HID = 256
_D_OUT = 128

_NC = 2
_NS = 16
_NW = _NC * _NS

_CH = 128
_NCHUNK_RAW = _E // _CH
_PER_W = -(-_NCHUNK_RAW // _NW)
_NCHUNK = _PER_W * _NW
_EP = _NCHUNK * _CH
_NP = 10016
_RPS = _NP // _NS


def _sc_agg_call(feat, srcp, dstp, zf, zd, ones_h, compute_deg):
    mesh = plsc.VectorSubcoreMesh(core_axis_name="c", subcore_axis_name="s")

    out_type = [jax.ShapeDtypeStruct((_NC, _NP, 128), jnp.float32)]
    scratch = [
        pltpu.VMEM_SHARED((_NP, 128), jnp.float32),
        pltpu.VMEM((_CH,), jnp.int32),
        pltpu.VMEM((_CH,), jnp.int32),
        pltpu.VMEM((_CH, 128), jnp.float32),
    ]
    if compute_deg:
        out_type.append(jax.ShapeDtypeStruct((_NC, _NP, 16), jnp.float32))
        scratch.append(pltpu.VMEM_SHARED((_NP, 16), jnp.float32))
        scratch.append(pltpu.VMEM((_CH, 16), jnp.float32))

    @functools.partial(
        pl.kernel,
        out_type=out_type,
        mesh=mesh,
        scratch_types=scratch,
    )
    def k(feat_hbm, src_hbm, dst_hbm, zf_hbm, zd_hbm, ones_hbm, *rest):
        if compute_deg:
            agg_out, deg_out, accum, src_v, dst_v, rows_v, dacc, ones_v = rest
        else:
            agg_out, accum, src_v, dst_v, rows_v = rest
        c = lax.axis_index("c")
        s = lax.axis_index("s")
        w = s * _NC + c
        r0 = s * _RPS

        pltpu.sync_copy(zf_hbm.at[pl.ds(r0, _RPS)], accum.at[pl.ds(r0, _RPS)])
        if compute_deg:
            pltpu.sync_copy(zd_hbm.at[pl.ds(r0, _RPS)], dacc.at[pl.ds(r0, _RPS)])
            pltpu.sync_copy(ones_hbm, ones_v)
        plsc.subcore_barrier()

        @pl.loop(0, _PER_W)
        def _(i):
            base = (w * _PER_W + i) * _CH
            pltpu.sync_copy(src_hbm.at[pl.ds(base, _CH)], src_v)
            pltpu.sync_copy(dst_hbm.at[pl.ds(base, _CH)], dst_v)
            pltpu.sync_copy(feat_hbm.at[src_v], rows_v)
            pltpu.sync_copy(rows_v, accum.at[dst_v], add=True)
            if compute_deg:
                pltpu.sync_copy(ones_v, dacc.at[dst_v], add=True)

        plsc.subcore_barrier()
        pltpu.sync_copy(accum.at[pl.ds(r0, _RPS)], agg_out.at[c, pl.ds(r0, _RPS)])
        if compute_deg:
            pltpu.sync_copy(dacc.at[pl.ds(r0, _RPS)], deg_out.at[c, pl.ds(r0, _RPS)])

    res = k(feat, srcp, dstp, zf, zd, ones_h)
    if compute_deg:
        return res[0], res[1]
    return res[0], None


_BLK = 1000


def _dot(a, b):
    return jax.lax.dot_general(
        a, b, (((1,), (0,)), ((), ())),
        precision=jax.lax.Precision.HIGHEST,
        preferred_element_type=jnp.float32)


def _dense1_body(x_r, a0_r, a1_r, d0_r, d1_r, ws1_r, wn1_r, b1_r, ws2_r,
                 wn2_r, b2_r, o1_r, o2_r):
    deg = d0_r[:, 0:1] + d1_r[:, 0:1]
    deginv = 1.0 / jnp.maximum(deg, 1.0)
    agg = (a0_r[...] + a1_r[...]) * deginv
    h1 = _dot(x_r[...], ws1_r[...]) + _dot(agg, wn1_r[...]) + b1_r[...]
    h1 = jnp.maximum(h1, 0.0)
    o1_r[...] = _dot(h1, ws2_r[...]) + b2_r[...]
    o2_r[...] = _dot(h1, wn2_r[...])


def _dense1(x, a0, a1, d0, d1, Ws1, Wn1, b1, Ws2, Wn2, b2):
    grid = (_N // _BLK,)
    row = lambda r: (r, 0)
    fixed = lambda r: (0, 0)
    return pl.pallas_call(
        _dense1_body,
        grid=grid,
        in_specs=[
            pl.BlockSpec((_BLK, _D_IN), row),
            pl.BlockSpec((_BLK, 128), row),
            pl.BlockSpec((_BLK, 128), row),
            pl.BlockSpec((_BLK, 16), row),
            pl.BlockSpec((_BLK, 16), row),
            pl.BlockSpec((_D_IN, _D_HID), fixed),
            pl.BlockSpec((_D_IN, _D_HID), fixed),
            pl.BlockSpec((1, _D_HID), fixed),
            pl.BlockSpec((_D_HID, _D_OUT), fixed),
            pl.BlockSpec((_D_HID, _D_OUT), fixed),
            pl.BlockSpec((1, _D_OUT), fixed),
        ],
        out_specs=[
            pl.BlockSpec((_BLK, _D_OUT), row),
            pl.BlockSpec((_BLK, _D_OUT), row),
        ],
        out_shape=[
            jax.ShapeDtypeStruct((_N, _D_OUT), jnp.float32),
            jax.ShapeDtypeStruct((_N, _D_OUT), jnp.float32),
        ],
    )(x, a0, a1, d0, d1, Ws1, Wn1, b1, Ws2, Wn2, b2)


def _dense2_body(o1_r, a0_r, a1_r, d0_r, d1_r, out_r):
    deg = d0_r[:, 0:1] + d1_r[:, 0:1]
    deginv = 1.0 / jnp.maximum(deg, 1.0)
    h2 = o1_r[...] + (a0_r[...] + a1_r[...]) * deginv
    h2 = jnp.maximum(h2, 0.0)
    norm = jnp.sqrt(jnp.sum(h2 * h2, axis=1, keepdims=True))
    out_r[...] = h2 / jnp.maximum(norm, 1e-12)


def _dense2(o1, a0, a1, d0, d1):
    grid = (_N // _BLK,)
    row = lambda r: (r, 0)
    return pl.pallas_call(
        _dense2_body,
        grid=grid,
        in_specs=[
            pl.BlockSpec((_BLK, _D_OUT), row),
            pl.BlockSpec((_BLK, 128), row),
            pl.BlockSpec((_BLK, 128), row),
            pl.BlockSpec((_BLK, 16), row),
            pl.BlockSpec((_BLK, 16), row),
        ],
        out_specs=pl.BlockSpec((_BLK, _D_OUT), row),
        out_shape=jax.ShapeDtypeStruct((_N, _D_OUT), jnp.float32),
    )(o1, a0, a1, d0, d1)


def kernel(in_feat, edge_index, W_self1, W_neigh1, b1, W_self2, W_neigh2, b2):
    src = edge_index[0]
    dst = edge_index[1]
    pad = _EP - _E
    srcp = jnp.concatenate([src, jnp.zeros((pad,), jnp.int32)])
    dstp = jnp.concatenate([dst, jnp.full((pad,), _N, jnp.int32)])
    zf = jnp.zeros((_NP, 128), jnp.float32)
    zd = jnp.zeros((_NP, 16), jnp.float32)
    ones_h = jnp.ones((_CH, 16), jnp.float32)

    agg1, degp = _sc_agg_call(in_feat, srcp, dstp, zf, zd, ones_h, True)
    a0, a1 = agg1[0, :_N], agg1[1, :_N]
    d0, d1 = degp[0, :_N], degp[1, :_N]
    o1, m2 = _dense1(in_feat, a0, a1, d0, d1, W_self1, W_neigh1,
                     b1.reshape(1, -1), W_self2, W_neigh2, b2.reshape(1, -1))
    agg2, _ = _sc_agg_call(m2, srcp, dstp, zf, zd, ones_h, False)
    return _dense2(o1, agg2[0, :_N], agg2[1, :_N], d0, d1)


# speedup vs baseline: 4.5273x; 4.5273x over previous
import functools

import jax
import jax.numpy as jnp
from jax import lax
from jax.experimental import pallas as pl
from jax.experimental.pallas import tpu as pltpu
from jax.experimental.pallas import tpu_sc as plsc

_N = 10000
_E = 320000
_D_IN = 128
_D_HID = 256
_D_OUT = 128

_NC = 2
_NS = 16
_NW = _NC * _NS

_CH = 128
_NCHUNK_RAW = _E // _CH
_PER_W = -(-_NCHUNK_RAW // _NW)
_NCHUNK = _PER_W * _NW
_EP = _NCHUNK * _CH
_NP = 10112
_RPS = _NP // _NS


def _sc_agg_call(feat, srcp, dstp, zf):
    mesh = plsc.VectorSubcoreMesh(core_axis_name="c", subcore_axis_name="s")

    @functools.partial(
        pl.kernel,
        out_type=jax.ShapeDtypeStruct((_NC, _NP, 128), jnp.float32),
        mesh=mesh,
        scratch_types=[
            pltpu.VMEM_SHARED((_NP, 128), jnp.float32),
            pltpu.VMEM((_CH,), jnp.int32),
            pltpu.VMEM((_CH,), jnp.int32),
            pltpu.VMEM((_CH, 128), jnp.float32),
        ],
    )
    def k(feat_hbm, src_hbm, dst_hbm, zf_hbm, agg_out, accum, src_v, dst_v,
          rows_v):
        c = lax.axis_index("c")
        s = lax.axis_index("s")
        w = s * _NC + c
        r0 = s * _RPS

        pltpu.sync_copy(zf_hbm.at[pl.ds(r0, _RPS)], accum.at[pl.ds(r0, _RPS)])
        plsc.subcore_barrier()

        @pl.loop(0, _PER_W)
        def _(i):
            base = (w * _PER_W + i) * _CH
            pltpu.sync_copy(src_hbm.at[pl.ds(base, _CH)], src_v)
            pltpu.sync_copy(dst_hbm.at[pl.ds(base, _CH)], dst_v)
            pltpu.sync_copy(feat_hbm.at[src_v], rows_v)
            pltpu.sync_copy(rows_v, accum.at[dst_v], add=True)

        plsc.subcore_barrier()
        pltpu.sync_copy(accum.at[pl.ds(r0, _RPS)], agg_out.at[c, pl.ds(r0, _RPS)])

    return k(feat, srcp, dstp, zf)


_BLK = 1000


def _dot(a, b):
    return jax.lax.dot_general(
        a, b, (((1,), (0,)), ((), ())),
        precision=jax.lax.Precision.HIGHEST,
        preferred_element_type=jnp.float32)


def _dense1_body(x_r, a0_r, a1_r, d0_r, d1_r, ws1_r, wn1_r, b1_r, ws2_r,
                 wn2_r, b2_r, o1_r, o2_r):
    deg = d0_r[:, 0:1] + d1_r[:, 0:1]
    deginv = 1.0 / jnp.maximum(deg, 1.0)
    agg = (a0_r[...] + a1_r[...]) * deginv
    h1 = _dot(x_r[...], ws1_r[...]) + _dot(agg, wn1_r[...]) + b1_r[...]
    h1 = jnp.maximum(h1, 0.0)
    o1_r[...] = _dot(h1, ws2_r[...]) + b2_r[...]
    o2_r[...] = _dot(h1, wn2_r[...])


def _dense1(x, a0, a1, d0, d1, Ws1, Wn1, b1, Ws2, Wn2, b2):
    grid = (_N // _BLK,)
    row = lambda r: (r, 0)
    fixed = lambda r: (0, 0)
    return pl.pallas_call(
        _dense1_body,
        grid=grid,
        in_specs=[
            pl.BlockSpec((_BLK, _D_IN), row),
            pl.BlockSpec((_BLK, 128), row),
            pl.BlockSpec((_BLK, 128), row),
            pl.BlockSpec((_BLK, 16), row),
            pl.BlockSpec((_BLK, 16), row),
            pl.BlockSpec((_D_IN, _D_HID), fixed),
            pl.BlockSpec((_D_IN, _D_HID), fixed),
            pl.BlockSpec((1, _D_HID), fixed),
            pl.BlockSpec((_D_HID, _D_OUT), fixed),
            pl.BlockSpec((_D_HID, _D_OUT), fixed),
            pl.BlockSpec((1, _D_OUT), fixed),
        ],
        out_specs=[
            pl.BlockSpec((_BLK, _D_OUT), row),
            pl.BlockSpec((_BLK, _D_OUT), row),
        ],
        out_shape=[
            jax.ShapeDtypeStruct((_N, _D_OUT), jnp.float32),
            jax.ShapeDtypeStruct((_N, _D_OUT), jnp.float32),
        ],
    )(x, a0, a1, d0, d1, Ws1, Wn1, b1, Ws2, Wn2, b2)


def _dense2_body(o1_r, a0_r, a1_r, d0_r, d1_r, out_r):
    deg = d0_r[:, 0:1] + d1_r[:, 0:1]
    deginv = 1.0 / jnp.maximum(deg, 1.0)
    h2 = o1_r[...] + (a0_r[...] + a1_r[...]) * deginv
    h2 = jnp.maximum(h2, 0.0)
    norm = jnp.sqrt(jnp.sum(h2 * h2, axis=1, keepdims=True))
    out_r[...] = h2 / jnp.maximum(norm, 1e-12)


def _dense2(o1, a0, a1, d0, d1):
    grid = (_N // _BLK,)
    row = lambda r: (r, 0)
    return pl.pallas_call(
        _dense2_body,
        grid=grid,
        in_specs=[
            pl.BlockSpec((_BLK, _D_OUT), row),
            pl.BlockSpec((_BLK, 128), row),
            pl.BlockSpec((_BLK, 128), row),
            pl.BlockSpec((_BLK, 16), row),
            pl.BlockSpec((_BLK, 16), row),
        ],
        out_specs=pl.BlockSpec((_BLK, _D_OUT), row),
        out_shape=jax.ShapeDtypeStruct((_N, _D_OUT), jnp.float32),
    )(o1, a0, a1, d0, d1)


def kernel(in_feat, edge_index, W_self1, W_neigh1, b1, W_self2, W_neigh2, b2):
    src = edge_index[0]
    dst = edge_index[1]
    pad = _EP - _E
    srcp = jnp.concatenate([src, jnp.zeros((pad,), jnp.int32)])
    pad_dst = _N + (jnp.arange(pad, dtype=jnp.int32) % (_NP - _N))
    dstp = jnp.concatenate([dst, pad_dst])
    zf = jnp.zeros((_NP, 128), jnp.float32)

    deg = jax.ops.segment_sum(jnp.ones((_E,), jnp.float32), dst,
                              num_segments=_N)
    d0 = jnp.tile(deg[:, None], (1, 16))
    d1 = jnp.zeros((_N, 16), jnp.float32)

    agg1 = _sc_agg_call(in_feat, srcp, dstp, zf)
    a0, a1 = agg1[0, :_N], agg1[1, :_N]
    o1, m2 = _dense1(in_feat, a0, a1, d0, d1, W_self1, W_neigh1,
                     b1.reshape(1, -1), W_self2, W_neigh2, b2.reshape(1, -1))
    agg2 = _sc_agg_call(m2, srcp, dstp, zf)
    return _dense2(o1, agg2[0, :_N], agg2[1, :_N], d0, d1)


# speedup vs baseline: 4.6771x; 1.0331x over previous
import dataclasses
import functools

import jax
import jax.numpy as jnp
from jax import lax
from jax.experimental import pallas as pl
from jax.experimental.pallas import tpu as pltpu
from jax.experimental.pallas import tpu_sc as plsc

_N = 10000
_E = 320000
_D_IN = 128
_D_HID = 256
_D_OUT = 128

_NC = 2
_NS = 16
_NW = _NC * _NS
_L = 16

_CH = 128
_NCHUNK_RAW = _E // _CH
_PER_W = -(-_NCHUNK_RAW // _NW)
_NCHUNK = _PER_W * _NW
_EP = _NCHUNK * _CH
_NP = 10112
_RPS = _NP // _NS


def _sc_agg_call(feat, srcp, dstp, zf, zv, compute_deg):
    mesh = plsc.VectorSubcoreMesh(core_axis_name="c", subcore_axis_name="s")

    out_type = [jax.ShapeDtypeStruct((_NC, _NP, 128), jnp.float32)]
    scratch = [
        pltpu.VMEM_SHARED((_NP, 128), jnp.float32),
        pltpu.VMEM((_CH,), jnp.int32),
        pltpu.VMEM((_CH,), jnp.int32),
        pltpu.VMEM((_CH, 128), jnp.float32),
    ]
    if compute_deg:
        out_type.append(jax.ShapeDtypeStruct((_NW, _NP), jnp.float32))
        scratch.append(pltpu.VMEM((_NP,), jnp.float32))

    cp = pltpu.CompilerParams()
    if compute_deg and (
            "needs_layout_passes" in pltpu.CompilerParams.__dataclass_fields__):
        cp = dataclasses.replace(cp, needs_layout_passes=False)

    @functools.partial(
        pl.kernel,
        out_type=out_type,
        mesh=mesh,
        scratch_types=scratch,
        compiler_params=cp,
    )
    def k(feat_hbm, src_hbm, dst_hbm, zf_hbm, zv_hbm, *rest):
        if compute_deg:
            agg_out, deg_out, accum, src_v, dst_v, rows_v, deg_v = rest
        else:
            agg_out, accum, src_v, dst_v, rows_v = rest
        c = lax.axis_index("c")
        s = lax.axis_index("s")
        w = s * _NC + c
        r0 = s * _RPS

        pltpu.sync_copy(zf_hbm.at[pl.ds(r0, _RPS)], accum.at[pl.ds(r0, _RPS)])
        if compute_deg:
            pltpu.sync_copy(zv_hbm, deg_v)
        plsc.subcore_barrier()

        ones16 = jnp.ones((_L,), jnp.float32)

        @pl.loop(0, _PER_W)
        def _(i):
            base = (w * _PER_W + i) * _CH
            pltpu.sync_copy(src_hbm.at[pl.ds(base, _CH)], src_v)
            pltpu.sync_copy(dst_hbm.at[pl.ds(base, _CH)], dst_v)
            pltpu.sync_copy(feat_hbm.at[src_v], rows_v)
            pltpu.sync_copy(rows_v, accum.at[dst_v], add=True)
            if compute_deg:
                for j in range(_CH // _L):
                    idx16 = dst_v[pl.ds(j * _L, _L)]
                    plsc.addupdate_scatter(deg_v, [idx16], ones16)

        plsc.subcore_barrier()
        pltpu.sync_copy(accum.at[pl.ds(r0, _RPS)], agg_out.at[c, pl.ds(r0, _RPS)])
        if compute_deg:
            pltpu.sync_copy(deg_v, deg_out.at[w])

    res = k(feat, srcp, dstp, zf, zv)
    if compute_deg:
        return res[0], res[1]
    return res[0], None


_BLK = 1000


def _dot(a, b):
    return jax.lax.dot_general(
        a, b, (((1,), (0,)), ((), ())),
        precision=jax.lax.Precision.HIGHEST,
        preferred_element_type=jnp.float32)


def _dense1_body(x_r, a0_r, a1_r, dt_r, ws1_r, wn1_r, b1_r, ws2_r,
                 wn2_r, b2_r, o1_r, o2_r):
    deg = jnp.sum(dt_r[...], axis=1, keepdims=True)
    deginv = 1.0 / jnp.maximum(deg, 1.0)
    agg = (a0_r[...] + a1_r[...]) * deginv
    h1 = _dot(x_r[...], ws1_r[...]) + _dot(agg, wn1_r[...]) + b1_r[...]
    h1 = jnp.maximum(h1, 0.0)
    o1_r[...] = _dot(h1, ws2_r[...]) + b2_r[...]
    o2_r[...] = _dot(h1, wn2_r[...])


def _dense1(x, a0, a1, dt, Ws1, Wn1, b1, Ws2, Wn2, b2):
    grid = (_N // _BLK,)
    row = lambda r: (r, 0)
    fixed = lambda r: (0, 0)
    return pl.pallas_call(
        _dense1_body,
        grid=grid,
        in_specs=[
            pl.BlockSpec((_BLK, _D_IN), row),
            pl.BlockSpec((_BLK, 128), row),
            pl.BlockSpec((_BLK, 128), row),
            pl.BlockSpec((_BLK, _NW), row),
            pl.BlockSpec((_D_IN, _D_HID), fixed),
            pl.BlockSpec((_D_IN, _D_HID), fixed),
            pl.BlockSpec((1, _D_HID), fixed),
            pl.BlockSpec((_D_HID, _D_OUT), fixed),
            pl.BlockSpec((_D_HID, _D_OUT), fixed),
            pl.BlockSpec((1, _D_OUT), fixed),
        ],
        out_specs=[
            pl.BlockSpec((_BLK, _D_OUT), row),
            pl.BlockSpec((_BLK, _D_OUT), row),
        ],
        out_shape=[
            jax.ShapeDtypeStruct((_N, _D_OUT), jnp.float32),
            jax.ShapeDtypeStruct((_N, _D_OUT), jnp.float32),
        ],
    )(x, a0, a1, dt, Ws1, Wn1, b1, Ws2, Wn2, b2)


def _dense2_body(o1_r, a0_r, a1_r, dt_r, out_r):
    deg = jnp.sum(dt_r[...], axis=1, keepdims=True)
    deginv = 1.0 / jnp.maximum(deg, 1.0)
    h2 = o1_r[...] + (a0_r[...] + a1_r[...]) * deginv
    h2 = jnp.maximum(h2, 0.0)
    norm = jnp.sqrt(jnp.sum(h2 * h2, axis=1, keepdims=True))
    out_r[...] = h2 / jnp.maximum(norm, 1e-12)


def _dense2(o1, a0, a1, dt):
    grid = (_N // _BLK,)
    row = lambda r: (r, 0)
    return pl.pallas_call(
        _dense2_body,
        grid=grid,
        in_specs=[
            pl.BlockSpec((_BLK, _D_OUT), row),
            pl.BlockSpec((_BLK, 128), row),
            pl.BlockSpec((_BLK, 128), row),
            pl.BlockSpec((_BLK, _NW), row),
        ],
        out_specs=pl.BlockSpec((_BLK, _D_OUT), row),
        out_shape=jax.ShapeDtypeStruct((_N, _D_OUT), jnp.float32),
    )(o1, a0, a1, dt)


def kernel(in_feat, edge_index, W_self1, W_neigh1, b1, W_self2, W_neigh2, b2):
    src = edge_index[0]
    dst = edge_index[1]
    pad = _EP - _E
    srcp = jnp.concatenate([src, jnp.zeros((pad,), jnp.int32)])
    pad_dst = _N + (jnp.arange(pad, dtype=jnp.int32) % (_NP - _N))
    dstp = jnp.concatenate([dst, pad_dst])
    zf = jnp.zeros((_NP, 128), jnp.float32)
    zv = jnp.zeros((_NP,), jnp.float32)

    agg1, degp = _sc_agg_call(in_feat, srcp, dstp, zf, zv, True)
    a0, a1 = agg1[0, :_N], agg1[1, :_N]
    dt = degp[:, :_N].T
    o1, m2 = _dense1(in_feat, a0, a1, dt, W_self1, W_neigh1,
                     b1.reshape(1, -1), W_self2, W_neigh2, b2.reshape(1, -1))
    agg2, _ = _sc_agg_call(m2, srcp, dstp, zf, zv, False)
    return _dense2(o1, agg2[0, :_N], agg2[1, :_N], dt)


# speedup vs baseline: 12.6139x; 2.6969x over previous
import dataclasses
import functools

import jax
import jax.numpy as jnp
from jax import lax
from jax.experimental import pallas as pl
from jax.experimental.pallas import tpu as pltpu
from jax.experimental.pallas import tpu_sc as plsc

_N = 10000
_E = 320000
_D_IN = 128
_D_HID = 256
_D_OUT = 128

_NC = 2
_NS = 16
_NW = _NC * _NS
_L = 16

_CH = 128
_SB = 8
_PER_W = 80
_NSB = _PER_W // _SB
_NCHUNK = _PER_W * _NW
_EP = _NCHUNK * _CH
_NP = 10112
_RPS = _NP // _NS


def _sc_agg_call(feat, src2, dst2, zf, zv, compute_deg):
    mesh = plsc.VectorSubcoreMesh(core_axis_name="c", subcore_axis_name="s")

    out_type = [jax.ShapeDtypeStruct((_NC, _NP, 128), jnp.float32)]
    scratch = [
        pltpu.VMEM_SHARED((_NP, 128), jnp.float32),
        pltpu.VMEM((_SB, _CH), jnp.int32),
        pltpu.VMEM((_SB, _CH), jnp.int32),
        pltpu.VMEM((_SB, _CH), jnp.int32),
        pltpu.VMEM((_SB, _CH), jnp.int32),
        pltpu.VMEM((_CH, 128), jnp.float32),
        pltpu.VMEM((_CH, 128), jnp.float32),
        pltpu.SemaphoreType.DMA,
        pltpu.SemaphoreType.DMA,
        pltpu.SemaphoreType.DMA,
        pltpu.SemaphoreType.DMA,
        pltpu.SemaphoreType.DMA,
        pltpu.SemaphoreType.DMA,
    ]
    if compute_deg:
        out_type.append(jax.ShapeDtypeStruct((_NW, _NP), jnp.float32))
        scratch.append(pltpu.VMEM((_NP,), jnp.float32))

    cp = pltpu.CompilerParams()
    if compute_deg and (
            "needs_layout_passes" in pltpu.CompilerParams.__dataclass_fields__):
        cp = dataclasses.replace(cp, needs_layout_passes=False)

    @functools.partial(
        pl.kernel,
        out_type=out_type,
        mesh=mesh,
        scratch_types=scratch,
        compiler_params=cp,
    )
    def k(feat_hbm, src_hbm, dst_hbm, zf_hbm, zv_hbm, *rest):
        if compute_deg:
            (agg_out, deg_out, accum, srcA, dstA, srcB, dstB, rows0, rows1,
             ias, iad, ibs, ibd, g0, g1, deg_v) = rest
        else:
            (agg_out, accum, srcA, dstA, srcB, dstB, rows0, rows1,
             ias, iad, ibs, ibd, g0, g1) = rest
            deg_v = None
        c = lax.axis_index("c")
        s = lax.axis_index("s")
        w = s * _NC + c
        r0 = s * _RPS
        rows = (rows0, rows1)
        gsem = (g0, g1)
        ones16 = jnp.ones((_L,), jnp.float32)

        pltpu.sync_copy(zf_hbm.at[pl.ds(r0, _RPS)], accum.at[pl.ds(r0, _RPS)])
        if compute_deg:
            pltpu.sync_copy(zv_hbm, deg_v)
        plsc.subcore_barrier()

        def sb_row(sb):
            return w * _PER_W + sb * _SB

        def fire_idx(sb, sbuf, dbuf, ssem, dsem):
            pltpu.async_copy(src_hbm.at[pl.ds(sb_row(sb), _SB)], sbuf, ssem)
            pltpu.async_copy(dst_hbm.at[pl.ds(sb_row(sb), _SB)], dbuf, dsem)

        def wait_idx(sb, sbuf, dbuf, ssem, dsem):
            pltpu.make_async_copy(
                src_hbm.at[pl.ds(sb_row(sb), _SB)], sbuf, ssem).wait()
            pltpu.make_async_copy(
                dst_hbm.at[pl.ds(sb_row(sb), _SB)], dbuf, dsem).wait()

        def fire_gather(sbuf, j, rbuf, sem):
            pltpu.async_copy(feat_hbm.at[sbuf.at[j]], rbuf, sem)

        def wait_gather(sbuf, j, rbuf, sem):
            pltpu.make_async_copy(feat_hbm.at[sbuf.at[j]], rbuf, sem).wait()

        def inner(sb, sbuf, dbuf, cross):
            for j in range(_SB):
                if j < _SB - 1:
                    fire_gather(sbuf, j + 1, rows[(j + 1) % 2], gsem[(j + 1) % 2])
                elif cross is not None:
                    cross()
                wait_gather(sbuf, j, rows[j % 2], gsem[j % 2])
                if compute_deg:
                    for l in range(_CH // _L):
                        idx16 = dbuf[j, pl.ds(l * _L, _L)]
                        plsc.addupdate_scatter(deg_v, [idx16], ones16)
                pltpu.sync_copy(rows[j % 2], accum.at[dbuf.at[j]], add=True)

        fire_idx(0, srcA, dstA, ias, iad)
        wait_idx(0, srcA, dstA, ias, iad)
        fire_idx(1, srcB, dstB, ibs, ibd)
        fire_gather(srcA, 0, rows0, g0)

        @pl.loop(0, _NSB // 2)
        def _(kk):
            sb_a = 2 * kk
            not_last = kk < _NSB // 2 - 1

            def cross_a():
                wait_idx(sb_a + 1, srcB, dstB, ibs, ibd)
                fire_gather(srcB, 0, rows0, g0)

            inner(sb_a, srcA, dstA, cross_a)

            @pl.when(not_last)
            def _():
                fire_idx(sb_a + 2, srcA, dstA, ias, iad)

            def cross_b():
                @pl.when(not_last)
                def _():
                    wait_idx(sb_a + 2, srcA, dstA, ias, iad)
                    fire_gather(srcA, 0, rows0, g0)

            inner(sb_a + 1, srcB, dstB, cross_b)

            @pl.when(not_last)
            def _():
                fire_idx(sb_a + 3, srcB, dstB, ibs, ibd)

        plsc.subcore_barrier()
        pltpu.sync_copy(accum.at[pl.ds(r0, _RPS)], agg_out.at[c, pl.ds(r0, _RPS)])
        if compute_deg:
            pltpu.sync_copy(deg_v, deg_out.at[w])

    res = k(feat, src2, dst2, zf, zv)
    if compute_deg:
        return res[0], res[1]
    return res[0], None


_BLK = 1000


def _dot(a, b):
    return jax.lax.dot_general(
        a, b, (((1,), (0,)), ((), ())),
        precision=jax.lax.Precision.HIGHEST,
        preferred_element_type=jnp.float32)


def _dense1_body(x_r, a0_r, a1_r, dt_r, ws1_r, wn1_r, b1_r, ws2_r,
                 wn2_r, b2_r, o1_r, o2_r):
    deg = jnp.sum(dt_r[...], axis=1, keepdims=True)
    deginv = 1.0 / jnp.maximum(deg, 1.0)
    agg = (a0_r[...] + a1_r[...]) * deginv
    h1 = _dot(x_r[...], ws1_r[...]) + _dot(agg, wn1_r[...]) + b1_r[...]
    h1 = jnp.maximum(h1, 0.0)
    o1_r[...] = _dot(h1, ws2_r[...]) + b2_r[...]
    o2_r[...] = _dot(h1, wn2_r[...])


def _dense1(x, a0, a1, dt, Ws1, Wn1, b1, Ws2, Wn2, b2):
    grid = (_N // _BLK,)
    row = lambda r: (r, 0)
    fixed = lambda r: (0, 0)
    return pl.pallas_call(
        _dense1_body,
        grid=grid,
        in_specs=[
            pl.BlockSpec((_BLK, _D_IN), row),
            pl.BlockSpec((_BLK, 128), row),
            pl.BlockSpec((_BLK, 128), row),
            pl.BlockSpec((_BLK, _NW), row),
            pl.BlockSpec((_D_IN, _D_HID), fixed),
            pl.BlockSpec((_D_IN, _D_HID), fixed),
            pl.BlockSpec((1, _D_HID), fixed),
            pl.BlockSpec((_D_HID, _D_OUT), fixed),
            pl.BlockSpec((_D_HID, _D_OUT), fixed),
            pl.BlockSpec((1, _D_OUT), fixed),
        ],
        out_specs=[
            pl.BlockSpec((_BLK, _D_OUT), row),
            pl.BlockSpec((_BLK, _D_OUT), row),
        ],
        out_shape=[
            jax.ShapeDtypeStruct((_N, _D_OUT), jnp.float32),
            jax.ShapeDtypeStruct((_N, _D_OUT), jnp.float32),
        ],
    )(x, a0, a1, dt, Ws1, Wn1, b1, Ws2, Wn2, b2)


def _dense2_body(o1_r, a0_r, a1_r, dt_r, out_r):
    deg = jnp.sum(dt_r[...], axis=1, keepdims=True)
    deginv = 1.0 / jnp.maximum(deg, 1.0)
    h2 = o1_r[...] + (a0_r[...] + a1_r[...]) * deginv
    h2 = jnp.maximum(h2, 0.0)
    norm = jnp.sqrt(jnp.sum(h2 * h2, axis=1, keepdims=True))
    out_r[...] = h2 / jnp.maximum(norm, 1e-12)


def _dense2(o1, a0, a1, dt):
    grid = (_N // _BLK,)
    row = lambda r: (r, 0)
    return pl.pallas_call(
        _dense2_body,
        grid=grid,
        in_specs=[
            pl.BlockSpec((_BLK, _D_OUT), row),
            pl.BlockSpec((_BLK, 128), row),
            pl.BlockSpec((_BLK, 128), row),
            pl.BlockSpec((_BLK, _NW), row),
        ],
        out_specs=pl.BlockSpec((_BLK, _D_OUT), row),
        out_shape=jax.ShapeDtypeStruct((_N, _D_OUT), jnp.float32),
    )(o1, a0, a1, dt)


def kernel(in_feat, edge_index, W_self1, W_neigh1, b1, W_self2, W_neigh2, b2):
    src = edge_index[0]
    dst = edge_index[1]
    pad = _EP - _E
    pad_src = jnp.arange(pad, dtype=jnp.int32) % _N
    pad_dst = _N + (jnp.arange(pad, dtype=jnp.int32) % (_NP - _N))
    src2 = jnp.concatenate([src, pad_src]).reshape(_NCHUNK, _CH)
    dst2 = jnp.concatenate([dst, pad_dst]).reshape(_NCHUNK, _CH)
    zf = jnp.zeros((_NP, 128), jnp.float32)
    zv = jnp.zeros((_NP,), jnp.float32)

    agg1, degp = _sc_agg_call(in_feat, src2, dst2, zf, zv, True)
    a0, a1 = agg1[0, :_N], agg1[1, :_N]
    dt = degp[:, :_N].T
    o1, m2 = _dense1(in_feat, a0, a1, dt, W_self1, W_neigh1,
                     b1.reshape(1, -1), W_self2, W_neigh2, b2.reshape(1, -1))
    agg2, _ = _sc_agg_call(m2, src2, dst2, zf, zv, False)
    return _dense2(o1, agg2[0, :_N], agg2[1, :_N], dt)


# speedup vs baseline: 14.6690x; 1.1629x over previous
import dataclasses
import functools

import jax
import jax.numpy as jnp
from jax import lax
from jax.experimental import pallas as pl
from jax.experimental.pallas import tpu as pltpu
from jax.experimental.pallas import tpu_sc as plsc

_N = 10000
_E = 320000
_D_IN = 128
_D_HID = 256
_D_OUT = 128

_NC = 2
_NS = 16
_NW = _NC * _NS
_L = 16

_CH = 128
_SB = 8
_PER_W = 80
_NSB = _PER_W // _SB
_NCHUNK = _PER_W * _NW
_EP = _NCHUNK * _CH
_NP = 10112
_RPS = _NP // _NS


def _sc_agg_call(feat, src2, dst2, zf, zv, compute_deg):
    mesh = plsc.VectorSubcoreMesh(core_axis_name="c", subcore_axis_name="s")

    out_type = [jax.ShapeDtypeStruct((_NC, _NP, 128), jnp.float32)]
    scratch = [
        pltpu.VMEM_SHARED((_NP, 128), jnp.float32),
        pltpu.VMEM((_SB, _CH), jnp.int32),
        pltpu.VMEM((_SB, _CH), jnp.int32),
        pltpu.VMEM((_SB, _CH), jnp.int32),
        pltpu.VMEM((_SB, _CH), jnp.int32),
        pltpu.VMEM((_CH, 128), jnp.float32),
        pltpu.VMEM((_CH, 128), jnp.float32),
        pltpu.SemaphoreType.DMA,
        pltpu.SemaphoreType.DMA,
        pltpu.SemaphoreType.DMA,
        pltpu.SemaphoreType.DMA,
        pltpu.SemaphoreType.DMA,
        pltpu.SemaphoreType.DMA,
    ]
    if compute_deg:
        out_type.append(jax.ShapeDtypeStruct((_NW, _NP), jnp.float32))
        scratch.append(pltpu.VMEM((_NP,), jnp.float32))

    cp = pltpu.CompilerParams()
    if compute_deg and (
            "needs_layout_passes" in pltpu.CompilerParams.__dataclass_fields__):
        cp = dataclasses.replace(cp, needs_layout_passes=False)

    @functools.partial(
        pl.kernel,
        out_type=out_type,
        mesh=mesh,
        scratch_types=scratch,
        compiler_params=cp,
    )
    def k(feat_hbm, src_hbm, dst_hbm, zf_hbm, zv_hbm, *rest):
        if compute_deg:
            (agg_out, deg_out, accum, srcA, dstA, srcB, dstB, rows0, rows1,
             ias, iad, ibs, ibd, g0, g1, deg_v) = rest
        else:
            (agg_out, accum, srcA, dstA, srcB, dstB, rows0, rows1,
             ias, iad, ibs, ibd, g0, g1) = rest
            deg_v = None
        c = lax.axis_index("c")
        s = lax.axis_index("s")
        w = s * _NC + c
        r0 = s * _RPS
        rows = (rows0, rows1)
        gsem = (g0, g1)
        ones16 = jnp.ones((_L,), jnp.float32)

        pltpu.sync_copy(zf_hbm.at[pl.ds(r0, _RPS)], accum.at[pl.ds(r0, _RPS)])
        if compute_deg:
            pltpu.sync_copy(zv_hbm, deg_v)
        plsc.subcore_barrier()

        def sb_row(sb):
            return w * _PER_W + sb * _SB

        def fire_idx(sb, sbuf, dbuf, ssem, dsem):
            pltpu.async_copy(src_hbm.at[pl.ds(sb_row(sb), _SB)], sbuf, ssem)
            pltpu.async_copy(dst_hbm.at[pl.ds(sb_row(sb), _SB)], dbuf, dsem)

        def wait_idx(sb, sbuf, dbuf, ssem, dsem):
            pltpu.make_async_copy(
                src_hbm.at[pl.ds(sb_row(sb), _SB)], sbuf, ssem).wait()
            pltpu.make_async_copy(
                dst_hbm.at[pl.ds(sb_row(sb), _SB)], dbuf, dsem).wait()

        def fire_gather(sbuf, j, rbuf, sem):
            pltpu.async_copy(feat_hbm.at[sbuf.at[j]], rbuf, sem)

        def wait_gather(sbuf, j, rbuf, sem):
            pltpu.make_async_copy(feat_hbm.at[sbuf.at[j]], rbuf, sem).wait()

        def inner(sb, sbuf, dbuf, cross):
            for j in range(_SB):
                if j < _SB - 1:
                    fire_gather(sbuf, j + 1, rows[(j + 1) % 2], gsem[(j + 1) % 2])
                elif cross is not None:
                    cross()
                wait_gather(sbuf, j, rows[j % 2], gsem[j % 2])
                if compute_deg:
                    for l in range(_CH // _L):
                        idx16 = dbuf[j, pl.ds(l * _L, _L)]
                        plsc.addupdate_scatter(deg_v, [idx16], ones16)
                pltpu.sync_copy(rows[j % 2], accum.at[dbuf.at[j]], add=True)

        fire_idx(0, srcA, dstA, ias, iad)
        wait_idx(0, srcA, dstA, ias, iad)
        fire_idx(1, srcB, dstB, ibs, ibd)
        fire_gather(srcA, 0, rows0, g0)

        @pl.loop(0, _NSB // 2)
        def _(kk):
            sb_a = 2 * kk
            not_last = kk < _NSB // 2 - 1

            def cross_a():
                wait_idx(sb_a + 1, srcB, dstB, ibs, ibd)
                fire_gather(srcB, 0, rows0, g0)

            inner(sb_a, srcA, dstA, cross_a)

            @pl.when(not_last)
            def _():
                fire_idx(sb_a + 2, srcA, dstA, ias, iad)

            def cross_b():
                @pl.when(not_last)
                def _():
                    wait_idx(sb_a + 2, srcA, dstA, ias, iad)
                    fire_gather(srcA, 0, rows0, g0)

            inner(sb_a + 1, srcB, dstB, cross_b)

            @pl.when(not_last)
            def _():
                fire_idx(sb_a + 3, srcB, dstB, ibs, ibd)

        plsc.subcore_barrier()
        pltpu.sync_copy(accum.at[pl.ds(r0, _RPS)], agg_out.at[c, pl.ds(r0, _RPS)])
        if compute_deg:
            pltpu.sync_copy(deg_v, deg_out.at[w])

    res = k(feat, src2, dst2, zf, zv)
    if compute_deg:
        return res[0], res[1]
    return res[0], None


_BLK = 1000


def _dot(a, b):
    return jax.lax.dot_general(
        a, b, (((1,), (0,)), ((), ())),
        preferred_element_type=jnp.float32)


def _deg_col(dp_r):
    return jnp.sum(dp_r[...], axis=1, keepdims=True)


def _dense1_body(x_r, a0_r, a1_r, dp_r, ws1_r, wn1_r, b1_r, ws2_r,
                 wn2_r, b2_r, o1_r, o2_r):
    deginv = 1.0 / jnp.maximum(_deg_col(dp_r), 1.0)
    agg = (a0_r[...] + a1_r[...]) * deginv
    h1 = _dot(x_r[...], ws1_r[...]) + _dot(agg, wn1_r[...]) + b1_r[...]
    h1 = jnp.maximum(h1, 0.0)
    o1_r[...] = _dot(h1, ws2_r[...]) + b2_r[...]
    o2_r[...] = _dot(h1, wn2_r[...])


def _dense1(x, a0, a1, dp, Ws1, Wn1, b1, Ws2, Wn2, b2):
    grid = (_N // _BLK,)
    row = lambda r: (r, 0)
    fixed = lambda r: (0, 0)
    return pl.pallas_call(
        _dense1_body,
        grid=grid,
        in_specs=[
            pl.BlockSpec((_BLK, _D_IN), row),
            pl.BlockSpec((_BLK, 128), row),
            pl.BlockSpec((_BLK, 128), row),
            pl.BlockSpec((_BLK, _NW), row),
            pl.BlockSpec((_D_IN, _D_HID), fixed),
            pl.BlockSpec((_D_IN, _D_HID), fixed),
            pl.BlockSpec((1, _D_HID), fixed),
            pl.BlockSpec((_D_HID, _D_OUT), fixed),
            pl.BlockSpec((_D_HID, _D_OUT), fixed),
            pl.BlockSpec((1, _D_OUT), fixed),
        ],
        out_specs=[
            pl.BlockSpec((_BLK, _D_OUT), row),
            pl.BlockSpec((_BLK, _D_OUT), row),
        ],
        out_shape=[
            jax.ShapeDtypeStruct((_N, _D_OUT), jnp.float32),
            jax.ShapeDtypeStruct((_N, _D_OUT), jnp.float32),
        ],
    )(x, a0, a1, dp, Ws1, Wn1, b1, Ws2, Wn2, b2)


def _dense2_body(o1_r, a0_r, a1_r, dp_r, out_r):
    deginv = 1.0 / jnp.maximum(_deg_col(dp_r), 1.0)
    h2 = o1_r[...] + (a0_r[...] + a1_r[...]) * deginv
    h2 = jnp.maximum(h2, 0.0)
    norm = jnp.sqrt(jnp.sum(h2 * h2, axis=1, keepdims=True))
    out_r[...] = h2 / jnp.maximum(norm, 1e-12)


def _dense2(o1, a0, a1, dp):
    grid = (_N // _BLK,)
    row = lambda r: (r, 0)
    return pl.pallas_call(
        _dense2_body,
        grid=grid,
        in_specs=[
            pl.BlockSpec((_BLK, _D_OUT), row),
            pl.BlockSpec((_BLK, 128), row),
            pl.BlockSpec((_BLK, 128), row),
            pl.BlockSpec((_BLK, _NW), row),
        ],
        out_specs=pl.BlockSpec((_BLK, _D_OUT), row),
        out_shape=jax.ShapeDtypeStruct((_N, _D_OUT), jnp.float32),
    )(o1, a0, a1, dp)


def kernel(in_feat, edge_index, W_self1, W_neigh1, b1, W_self2, W_neigh2, b2):
    src = edge_index[0]
    dst = edge_index[1]
    pad = _EP - _E
    pad_src = jnp.arange(pad, dtype=jnp.int32) % _N
    pad_dst = _N + (jnp.arange(pad, dtype=jnp.int32) % (_NP - _N))
    src2 = jnp.concatenate([src, pad_src]).reshape(_NCHUNK, _CH)
    dst2 = jnp.concatenate([dst, pad_dst]).reshape(_NCHUNK, _CH)
    zf = jnp.zeros((_NP, 128), jnp.float32)
    zv = jnp.zeros((_NP,), jnp.float32)

    agg1, degp = _sc_agg_call(in_feat, src2, dst2, zf, zv, True)
    a0, a1 = agg1[0, :_N], agg1[1, :_N]
    dp = degp[:, :_N].T
    o1, m2 = _dense1(in_feat, a0, a1, dp, W_self1, W_neigh1,
                     b1.reshape(1, -1), W_self2, W_neigh2, b2.reshape(1, -1))
    agg2, _ = _sc_agg_call(m2, src2, dst2, zf, zv, False)
    return _dense2(o1, agg2[0, :_N], agg2[1, :_N], dp)


# speedup vs baseline: 15.3146x; 1.0440x over previous
import dataclasses
import functools

import jax
import jax.numpy as jnp
from jax import lax
from jax.experimental import pallas as pl
from jax.experimental.pallas import tpu as pltpu
from jax.experimental.pallas import tpu_sc as plsc

_N = 10000
_E = 320000
_D_IN = 128
_D_HID = 256
_D_OUT = 128

_NC = 2
_NS = 16
_NW = _NC * _NS
_L = 16

_CH = 128
_SB = 8
_PER_W = 80
_NSB = _PER_W // _SB
_NCHUNK = _PER_W * _NW
_EP = _NCHUNK * _CH
_NP = 10112
_RPS = _NP // _NS


def _sc_agg_call(feat, src2, dst2, zf, zv, compute_deg):
    mesh = plsc.VectorSubcoreMesh(core_axis_name="c", subcore_axis_name="s")

    out_type = [jax.ShapeDtypeStruct((_NC, _NP, 128), jnp.float32)]
    scratch = [
        pltpu.VMEM_SHARED((_NP, 128), jnp.float32),
        pltpu.VMEM((_SB, _CH), jnp.int32),
        pltpu.VMEM((_SB, _CH), jnp.int32),
        pltpu.VMEM((_SB, _CH), jnp.int32),
        pltpu.VMEM((_SB, _CH), jnp.int32),
        pltpu.VMEM((_CH, 128), jnp.float32),
        pltpu.VMEM((_CH, 128), jnp.float32),
        pltpu.SemaphoreType.DMA,
        pltpu.SemaphoreType.DMA,
        pltpu.SemaphoreType.DMA,
        pltpu.SemaphoreType.DMA,
        pltpu.SemaphoreType.DMA,
        pltpu.SemaphoreType.DMA,
        pltpu.SemaphoreType.DMA,
        pltpu.SemaphoreType.DMA,
    ]
    if compute_deg:
        out_type.append(jax.ShapeDtypeStruct((_NW, _NP), jnp.float32))
        scratch.append(pltpu.VMEM((_NP,), jnp.float32))

    cp = pltpu.CompilerParams()
    if compute_deg and (
            "needs_layout_passes" in pltpu.CompilerParams.__dataclass_fields__):
        cp = dataclasses.replace(cp, needs_layout_passes=False)

    @functools.partial(
        pl.kernel,
        out_type=out_type,
        mesh=mesh,
        scratch_types=scratch,
        compiler_params=cp,
    )
    def k(feat_hbm, src_hbm, dst_hbm, zf_hbm, zv_hbm, *rest):
        if compute_deg:
            (agg_out, deg_out, accum, srcA, dstA, srcB, dstB, rows0, rows1,
             ias, iad, ibs, ibd, g0, g1, sc0, sc1, deg_v) = rest
        else:
            (agg_out, accum, srcA, dstA, srcB, dstB, rows0, rows1,
             ias, iad, ibs, ibd, g0, g1, sc0, sc1) = rest
            deg_v = None
        c = lax.axis_index("c")
        s = lax.axis_index("s")
        w = s * _NC + c
        r0 = s * _RPS
        rows = (rows0, rows1)
        gsem = (g0, g1)
        scsem = (sc0, sc1)
        ones16 = jnp.ones((_L,), jnp.float32)

        pltpu.sync_copy(zf_hbm.at[pl.ds(r0, _RPS)], accum.at[pl.ds(r0, _RPS)])
        if compute_deg:
            pltpu.sync_copy(zv_hbm, deg_v)
        plsc.subcore_barrier()

        def sb_row(sb):
            return w * _PER_W + sb * _SB

        def fire_idx(sb, sbuf, dbuf, ssem, dsem):
            pltpu.async_copy(src_hbm.at[pl.ds(sb_row(sb), _SB)], sbuf, ssem)
            pltpu.async_copy(dst_hbm.at[pl.ds(sb_row(sb), _SB)], dbuf, dsem)

        def wait_idx(sb, sbuf, dbuf, ssem, dsem):
            pltpu.make_async_copy(
                src_hbm.at[pl.ds(sb_row(sb), _SB)], sbuf, ssem).wait()
            pltpu.make_async_copy(
                dst_hbm.at[pl.ds(sb_row(sb), _SB)], dbuf, dsem).wait()

        def fire_gather(sbuf, j, rbuf, sem):
            pltpu.async_copy(feat_hbm.at[sbuf.at[j]], rbuf, sem)

        def wait_gather(sbuf, j, rbuf, sem):
            pltpu.make_async_copy(feat_hbm.at[sbuf.at[j]], rbuf, sem).wait()

        def fire_scatter(dbuf, j, par):
            pltpu.async_copy(rows[par], accum.at[dbuf.at[j]], scsem[par],
                             add=True)

        def wait_scatter(dbuf, j, par):
            pltpu.make_async_copy(rows[par], accum.at[dbuf.at[j]],
                                  scsem[par]).wait()

        def inner(sb, sbuf, dbuf, cross, first_wait):
            for j in range(_SB):
                if j < _SB - 1:
                    par = (j + 1) % 2
                    if first_wait is None and j == 0:
                        pass
                    elif j == 0:
                        first_wait(par)
                    else:
                        wait_scatter(dbuf, j - 1, par)
                    fire_gather(sbuf, j + 1, rows[par], gsem[par])
                elif cross is not None:
                    wait_scatter(dbuf, j - 1, 0)
                    cross()
                wait_gather(sbuf, j, rows[j % 2], gsem[j % 2])
                if compute_deg:
                    for l in range(_CH // _L):
                        idx16 = dbuf[j, pl.ds(l * _L, _L)]
                        plsc.addupdate_scatter(deg_v, [idx16], ones16)
                fire_scatter(dbuf, j, j % 2)

        fire_idx(0, srcA, dstA, ias, iad)
        wait_idx(0, srcA, dstA, ias, iad)
        fire_idx(1, srcB, dstB, ibs, ibd)
        fire_gather(srcA, 0, rows0, g0)

        @pl.loop(0, _NSB // 2)
        def _(kk):
            sb_a = 2 * kk
            not_last = kk < _NSB // 2 - 1
            not_first = kk > 0

            def first_wait_a(par):
                @pl.when(not_first)
                def _():
                    wait_scatter(srcB, _SB - 1, par)

            def first_wait_b(par):
                wait_scatter(srcA, _SB - 1, par)

            def cross_a():
                wait_idx(sb_a + 1, srcB, dstB, ibs, ibd)
                fire_gather(srcB, 0, rows0, g0)

            inner(sb_a, srcA, dstA, cross_a, first_wait_a)

            @pl.when(not_last)
            def _():
                fire_idx(sb_a + 2, srcA, dstA, ias, iad)

            def cross_b():
                @pl.when(not_last)
                def _():
                    wait_idx(sb_a + 2, srcA, dstA, ias, iad)
                    fire_gather(srcA, 0, rows0, g0)

            inner(sb_a + 1, srcB, dstB, cross_b, first_wait_b)

            @pl.when(not_last)
            def _():
                fire_idx(sb_a + 3, srcB, dstB, ibs, ibd)

        wait_scatter(srcB, _SB - 1, 1)
        plsc.subcore_barrier()
        pltpu.sync_copy(accum.at[pl.ds(r0, _RPS)], agg_out.at[c, pl.ds(r0, _RPS)])
        if compute_deg:
            pltpu.sync_copy(deg_v, deg_out.at[w])

    res = k(feat, src2, dst2, zf, zv)
    if compute_deg:
        return res[0], res[1]
    return res[0], None


_BLK = 1000


def _dot(a, b):
    return jax.lax.dot_general(
        a, b, (((1,), (0,)), ((), ())),
        preferred_element_type=jnp.float32)


def _deg_col(dp_r):
    return jnp.sum(dp_r[...], axis=1, keepdims=True)


def _dense1_body(x_r, a_r, dp_r, ws1_r, wn1_r, b1_r, ws2_r,
                 wn2_r, b2_r, o1_r, o2_r):
    deginv = 1.0 / jnp.maximum(_deg_col(dp_r), 1.0)
    agg = (a_r[0] + a_r[1]) * deginv
    h1 = _dot(x_r[...], ws1_r[...]) + _dot(agg, wn1_r[...]) + b1_r[...]
    h1 = jnp.maximum(h1, 0.0)
    o1_r[...] = _dot(h1, ws2_r[...]) + b2_r[...]
    o2_r[...] = _dot(h1, wn2_r[...])


def _dense1(x, aggp, dp, Ws1, Wn1, b1, Ws2, Wn2, b2):
    grid = (_N // _BLK,)
    row = lambda r: (r, 0)
    row3 = lambda r: (0, r, 0)
    fixed = lambda r: (0, 0)
    return pl.pallas_call(
        _dense1_body,
        grid=grid,
        in_specs=[
            pl.BlockSpec((_BLK, _D_IN), row),
            pl.BlockSpec((_NC, _BLK, 128), row3),
            pl.BlockSpec((_BLK, _NW), row),
            pl.BlockSpec((_D_IN, _D_HID), fixed),
            pl.BlockSpec((_D_IN, _D_HID), fixed),
            pl.BlockSpec((1, _D_HID), fixed),
            pl.BlockSpec((_D_HID, _D_OUT), fixed),
            pl.BlockSpec((_D_HID, _D_OUT), fixed),
            pl.BlockSpec((1, _D_OUT), fixed),
        ],
        out_specs=[
            pl.BlockSpec((_BLK, _D_OUT), row),
            pl.BlockSpec((_BLK, _D_OUT), row),
        ],
        out_shape=[
            jax.ShapeDtypeStruct((_N, _D_OUT), jnp.float32),
            jax.ShapeDtypeStruct((_N, _D_OUT), jnp.float32),
        ],
    )(x, aggp, dp, Ws1, Wn1, b1, Ws2, Wn2, b2)


def _dense2_body(o1_r, a_r, dp_r, out_r):
    deginv = 1.0 / jnp.maximum(_deg_col(dp_r), 1.0)
    h2 = o1_r[...] + (a_r[0] + a_r[1]) * deginv
    h2 = jnp.maximum(h2, 0.0)
    norm = jnp.sqrt(jnp.sum(h2 * h2, axis=1, keepdims=True))
    out_r[...] = h2 / jnp.maximum(norm, 1e-12)


def _dense2(o1, aggp, dp):
    grid = (_N // _BLK,)
    row = lambda r: (r, 0)
    row3 = lambda r: (0, r, 0)
    return pl.pallas_call(
        _dense2_body,
        grid=grid,
        in_specs=[
            pl.BlockSpec((_BLK, _D_OUT), row),
            pl.BlockSpec((_NC, _BLK, 128), row3),
            pl.BlockSpec((_BLK, _NW), row),
        ],
        out_specs=pl.BlockSpec((_BLK, _D_OUT), row),
        out_shape=jax.ShapeDtypeStruct((_N, _D_OUT), jnp.float32),
    )(o1, aggp, dp)


def kernel(in_feat, edge_index, W_self1, W_neigh1, b1, W_self2, W_neigh2, b2):
    src = edge_index[0]
    dst = edge_index[1]
    pad = _EP - _E
    pad_src = jnp.arange(pad, dtype=jnp.int32) % _N
    pad_dst = _N + (jnp.arange(pad, dtype=jnp.int32) % (_NP - _N))
    src2 = jnp.concatenate([src, pad_src]).reshape(_NCHUNK, _CH)
    dst2 = jnp.concatenate([dst, pad_dst]).reshape(_NCHUNK, _CH)
    zf = jnp.zeros((_NP, 128), jnp.float32)
    zv = jnp.zeros((_NP,), jnp.float32)

    agg1, degp = _sc_agg_call(in_feat, src2, dst2, zf, zv, True)
    dp = degp[:, :_N].T
    o1, m2 = _dense1(in_feat, agg1, dp, W_self1, W_neigh1,
                     b1.reshape(1, -1), W_self2, W_neigh2, b2.reshape(1, -1))
    agg2, _ = _sc_agg_call(m2, src2, dst2, zf, zv, False)
    return _dense2(o1, agg2, dp)
